# baseline (speedup 1.0000x reference)
import functools

import jax
import jax.numpy as jnp
from jax import lax
from jax.experimental import pallas as pl
from jax.experimental.pallas import tpu as pltpu
from jax.experimental.pallas import tpu_sc as plsc

B = 16384
H = 20
D = 64
NC = 2
NS = 16
NW = NC * NS
BPW = B // NW
IVL = 128
GPW = BPW // IVL
LANES = 16


def _body(idxt_hbm, table_hbm, out_hbm, idx_v, acc_v, acct_v, sem):
    c = lax.axis_index("c")
    s = lax.axis_index("s")
    wid = s * NC + c
    b0 = wid * BPW
    pltpu.sync_copy(idxt_hbm.at[:, pl.ds(b0, BPW)], idx_v)

    cps = []
    for j in range(GPW):
        cps.append(pltpu.async_copy(
            table_hbm.at[idx_v.at[0, pl.ds(j * IVL, IVL)]],
            acc_v.at[pl.ds(j * IVL, IVL)],
            sem,
        ))
    for cp in cps:
        cp.wait()

    def add_pass(l, carry):
        cps2 = []
        for j in range(GPW):
            cps2.append(pltpu.async_copy(
                table_hbm.at[idx_v.at[l, pl.ds(j * IVL, IVL)]],
                acc_v.at[pl.ds(j * IVL, IVL)],
                sem,
                add=True,
            ))
        for cp in cps2:
            cp.wait()
        return carry

    lax.fori_loop(1, H, add_pass, 0, unroll=False)

    lane = lax.iota(jnp.int32, LANES)

    def tr_col(d, carry):
        col = jnp.full((LANES,), d, jnp.int32)
        for vb in range(BPW // LANES):
            rows = lane + vb * LANES
            vals = plsc.load_gather(acc_v, [rows, col])
            acct_v[d, pl.ds(vb * LANES, LANES)] = vals * (1.0 / H)
        return carry

    lax.fori_loop(0, D, tr_col, 0, unroll=False)
    pltpu.sync_copy(acct_v, out_hbm.at[:, pl.ds(b0, BPW)])


_mesh = plsc.VectorSubcoreMesh(core_axis_name="c", subcore_axis_name="s")

_sc_call = functools.partial(
    pl.kernel,
    out_type=jax.ShapeDtypeStruct((D, B), jnp.float32),
    mesh=_mesh,
    scratch_types=[
        pltpu.VMEM((H, BPW), jnp.int32),
        pltpu.VMEM((BPW, D), jnp.float32),
        pltpu.VMEM((D, BPW), jnp.float32),
        pltpu.SemaphoreType.DMA,
    ],
    compiler_params=pltpu.CompilerParams(
        use_tc_tiling_on_sc=False, needs_layout_passes=False),
)(_body)


def kernel(indices, table):
    return _sc_call(indices.astype(jnp.int32).T, table).T


# speedup vs baseline: 1.1985x; 1.1985x over previous
import functools

import jax
import jax.numpy as jnp
from jax import lax
from jax.experimental import pallas as pl
from jax.experimental.pallas import tpu as pltpu
from jax.experimental.pallas import tpu_sc as plsc

B = 16384
H = 20
D = 64
NC = 2
NS = 16
NW = NC * NS
BPW = B // NW
IVL = 128
GPW = BPW // IVL
LANES = 16


def _body(idxt_hbm, table_hbm, out_hbm, idx_v, acc_v, sem):
    c = lax.axis_index("c")
    s = lax.axis_index("s")
    wid = s * NC + c
    b0 = wid * BPW
    pltpu.sync_copy(idxt_hbm.at[:, pl.ds(b0, BPW)], idx_v)

    cps = []
    for j in range(GPW):
        cps.append(pltpu.async_copy(
            table_hbm.at[idx_v.at[0, pl.ds(j * IVL, IVL)]],
            acc_v.at[pl.ds(j * IVL, IVL)],
            sem,
        ))
    for cp in cps:
        cp.wait()

    def add_pass(l, carry):
        for j in range(GPW):
            pltpu.async_copy(
                table_hbm.at[idx_v.at[l, pl.ds(j * IVL, IVL)]],
                acc_v.at[pl.ds(j * IVL, IVL)],
                sem,
                add=True,
            )
        return carry

    lax.fori_loop(1, H, add_pass, 0, unroll=False)

    def drain_pass(l, carry):
        for j in range(GPW):
            pltpu.make_async_copy(
                table_hbm.at[idx_v.at[0, pl.ds(j * IVL, IVL)]],
                acc_v.at[pl.ds(j * IVL, IVL)],
                sem,
            ).wait()
        return carry

    lax.fori_loop(1, H, drain_pass, 0, unroll=False)

    def scale_row(r, carry):
        for j in range(D // LANES):
            acc_v[r, pl.ds(j * LANES, LANES)] = (
                acc_v[r, pl.ds(j * LANES, LANES)] * (1.0 / H))
        return carry

    lax.fori_loop(0, BPW, scale_row, 0, unroll=False)
    pltpu.sync_copy(acc_v, out_hbm.at[pl.ds(b0, BPW)])


_mesh = plsc.VectorSubcoreMesh(core_axis_name="c", subcore_axis_name="s")

_sc_call = functools.partial(
    pl.kernel,
    out_type=jax.ShapeDtypeStruct((B, D), jnp.float32),
    mesh=_mesh,
    scratch_types=[
        pltpu.VMEM((H, BPW), jnp.int32),
        pltpu.VMEM((BPW, D), jnp.float32),
        pltpu.SemaphoreType.DMA,
    ],
    compiler_params=pltpu.CompilerParams(
        use_tc_tiling_on_sc=False, needs_layout_passes=False),
)(_body)


def kernel(indices, table):
    return _sc_call(indices.astype(jnp.int32).T, table)
